# baseline (speedup 1.0000x reference)
import functools

import jax
import jax.numpy as jnp
from jax import lax
from jax.experimental import pallas as pl
from jax.experimental.pallas import tpu as pltpu
from jax.experimental.pallas import tpu_sc as plsc

_LANES = 16
_N_SUBCORES = 16


def _gather_last_nodes(features, n_node):
    B = n_node.shape[0]
    D = features.shape[1]
    n_chunks = B // _LANES
    rows_per = B // _N_SUBCORES
    mesh = plsc.VectorSubcoreMesh(
        core_axis_name="c", subcore_axis_name="s", num_cores=1)

    @functools.partial(
        pl.kernel,
        out_type=jax.ShapeDtypeStruct((B, D), features.dtype),
        scratch_types=[
            pltpu.VMEM((B,), jnp.int32),
            pltpu.VMEM((B,), jnp.int32),
            pltpu.VMEM((rows_per, D), jnp.float32),
            pltpu.SemaphoreType.DMA,
        ],
        mesh=mesh,
    )
    def body(features_hbm, n_node_hbm, out_hbm, nn_v, idx_v, rows_v, sem):
        wid = lax.axis_index("s") + lax.axis_index("c")
        pltpu.sync_copy(n_node_hbm, nn_v)
        lanes = lax.iota(jnp.int32, _LANES)
        last = jnp.full((_LANES,), _LANES - 1, jnp.int32)
        carry = jnp.full((_LANES,), -1, jnp.int32)
        for i in range(n_chunks):
            v = nn_v[pl.ds(i * _LANES, _LANES)]
            for k in (1, 2, 4, 8):
                shifted = v.at[jnp.maximum(lanes - k, 0)].get(
                    mode="promise_in_bounds")
                v = v + jnp.where(lanes >= k, shifted, 0)
            v = v + carry
            idx_v[pl.ds(i * _LANES, _LANES)] = v
            carry = v.at[last].get(mode="promise_in_bounds")
        base = wid * rows_per
        pltpu.async_copy(
            features_hbm.at[idx_v.at[pl.ds(base, rows_per)]], rows_v, sem
        ).wait()
        pltpu.sync_copy(rows_v, out_hbm.at[pl.ds(base, rows_per)])

    return body(features, n_node)


def kernel(features, n_node, n_edge, globals, edges, senders, receivers):
    n_node = jnp.reshape(n_node, (-1,)).astype(jnp.int32)
    return _gather_last_nodes(features, n_node)


# speedup vs baseline: 1.0104x; 1.0104x over previous
import functools

import jax
import jax.numpy as jnp
from jax import lax
from jax.experimental import pallas as pl
from jax.experimental.pallas import tpu as pltpu
from jax.experimental.pallas import tpu_sc as plsc

_LANES = 16


def _gather_last_nodes(features, n_node):
    B = n_node.shape[0]
    D = features.shape[1]
    n_chunks = B // _LANES
    mesh = plsc.VectorSubcoreMesh(
        core_axis_name="c", subcore_axis_name="s", num_cores=1)

    @functools.partial(
        pl.kernel,
        out_type=jax.ShapeDtypeStruct((B, D), features.dtype),
        scratch_types=[
            pltpu.VMEM((B,), jnp.int32),
            pltpu.VMEM((_LANES, D), jnp.float32),
            pltpu.SemaphoreType.DMA,
        ],
        mesh=mesh,
    )
    def body(features_hbm, n_node_hbm, out_hbm, nn_v, rows_v, sem):
        wid = lax.axis_index("s") + lax.axis_index("c")

        @pl.when(wid < n_chunks)
        def _():
            pltpu.sync_copy(n_node_hbm, nn_v)
            lanes = lax.iota(jnp.int32, _LANES)
            last = jnp.full((_LANES,), _LANES - 1, jnp.int32)
            carry = jnp.full((_LANES,), -1, jnp.int32)
            my_idx = carry
            for i in range(n_chunks):
                v = nn_v[pl.ds(i * _LANES, _LANES)]
                for k in (1, 2, 4, 8):
                    shifted = v.at[jnp.maximum(lanes - k, 0)].get(
                        mode="promise_in_bounds")
                    v = v + jnp.where(lanes >= k, shifted, 0)
                v = v + carry
                my_idx = jnp.where(wid == i, v, my_idx)
                carry = v.at[last].get(mode="promise_in_bounds")
            pltpu.async_copy(features_hbm.at[my_idx], rows_v, sem).wait()
            pltpu.sync_copy(rows_v, out_hbm.at[pl.ds(wid * _LANES, _LANES)])

    return body(features, n_node)


def kernel(features, n_node, n_edge, globals, edges, senders, receivers):
    n_node = jnp.reshape(n_node, (-1,)).astype(jnp.int32)
    return _gather_last_nodes(features, n_node)
